# baseline (speedup 1.0000x reference)
import functools

import jax
import jax.numpy as jnp
from jax import lax
from jax.experimental import pallas as pl
from jax.experimental.pallas import tpu as pltpu
from jax.experimental.pallas import tpu_sc as plsc

_NC = 2
_NS = 16
_NW = _NC * _NS
_LANES = 16


def _rne_bf16_bits(x):
    u = lax.bitcast_convert_type(x, jnp.uint32)
    return u + jnp.uint32(0x7FFF) + ((u >> jnp.uint32(16)) & jnp.uint32(1))


def _proj_pack_body(w_ref, ht_ref, o_ref):
    ps = w_ref[10] * jnp.ones_like(ht_ref[0:1, :])
    pd = jnp.zeros_like(ps)
    for d in range(5):
        row = ht_ref[d : d + 1, :]
        ps = ps + w_ref[d] * row
        pd = pd + w_ref[5 + d] * row
    lo = _rne_bf16_bits(ps) >> jnp.uint32(16)
    hi = _rne_bf16_bits(pd) & jnp.uint32(0xFFFF0000)
    o_ref[...] = lax.bitcast_convert_type(hi | lo, jnp.int32).reshape(-1)


def _build_table(h, W, b, npad):
    n = h.shape[0]
    ht = jnp.zeros((8, npad), jnp.float32).at[:5, :n].set(h.T)
    wsm = jnp.concatenate(
        [W.reshape(-1), b.reshape(-1), jnp.zeros((5,), jnp.float32)]
    )
    return pl.pallas_call(
        _proj_pack_body,
        out_shape=jax.ShapeDtypeStruct((npad,), jnp.int32),
        in_specs=[
            pl.BlockSpec(memory_space=pltpu.SMEM),
            pl.BlockSpec(memory_space=pltpu.VMEM),
        ],
        out_specs=pl.BlockSpec(memory_space=pltpu.VMEM),
    )(wsm, ht)


def _make_sc_gather(e_total, npad, chunk):
    tot_chunks = e_total // chunk
    depth = 3
    n_t = -(-tot_chunks // _NW)
    n_t = -(-n_t // depth) * depth
    n_rounds = n_t // depth
    n_vec = chunk // _LANES
    mesh = plsc.VectorSubcoreMesh(core_axis_name="c", subcore_axis_name="s")

    @functools.partial(
        pl.kernel,
        mesh=mesh,
        compiler_params=pltpu.CompilerParams(needs_layout_passes=False),
        out_type=jax.ShapeDtypeStruct((e_total,), jnp.float32),
        scratch_types=(
            [pltpu.VMEM((npad,), jnp.int32)]
            + [pltpu.VMEM((2, chunk), jnp.int32) for _ in range(depth)]
            + [pltpu.VMEM((chunk,), jnp.float32) for _ in range(depth)]
            + [pltpu.SemaphoreType.DMA for _ in range(2 * depth + 1)]
        ),
    )
    def sc_kernel(ei_hbm, tbl_hbm, out_hbm, tbl_v, *bufs_flat):
        idx_bufs = bufs_flat[:depth]
        out_bufs = bufs_flat[depth:2 * depth]
        in_sems = bufs_flat[2 * depth:3 * depth]
        out_sems = bufs_flat[3 * depth:4 * depth]
        tbl_sem = bufs_flat[4 * depth]
        wid = lax.axis_index("s") * _NC + lax.axis_index("c")

        def chunk_off(j):
            return lax.rem(wid + _NW * j, tot_chunks) * chunk

        def start_in(j, b):
            pltpu.async_copy(ei_hbm.at[:, pl.ds(chunk_off(j), chunk)],
                             idx_bufs[b], in_sems[b])

        def wait_in(b):
            pltpu.make_async_copy(ei_hbm.at[:, pl.ds(0, chunk)],
                                  idx_bufs[b], in_sems[b]).wait()

        def wait_out(b):
            pltpu.make_async_copy(out_bufs[b], out_hbm.at[pl.ds(0, chunk)],
                                  out_sems[b]).wait()

        def compute(j, b):
            idx_v, out_v = idx_bufs[b], out_bufs[b]

            def vec_body(i):
                s = idx_v[0, pl.ds(i * _LANES, _LANES)]
                d = idx_v[1, pl.ds(i * _LANES, _LANES)]
                g_s = plsc.load_gather(tbl_v, [s])
                g_d = plsc.load_gather(tbl_v, [d])
                ps = plsc.bitcast(g_s << jnp.int32(16), jnp.float32)
                pd = plsc.bitcast(g_d & jnp.int32(-65536), jnp.float32)
                out_v[pl.ds(i * _LANES, _LANES)] = ps + pd

            plsc.parallel_loop(0, n_vec, 1, unroll=16)(vec_body)
            pltpu.async_copy(out_v, out_hbm.at[pl.ds(chunk_off(j), chunk)],
                             out_sems[b])

        pltpu.async_copy(tbl_hbm, tbl_v, tbl_sem)
        for b in range(depth - 1):
            start_in(b, b)
        pltpu.make_async_copy(tbl_hbm, tbl_v, tbl_sem).wait()

        def round_body(t, carry):
            for b in range(depth):
                j = depth * t + b
                jn = j + depth - 1
                pl.when(jn < n_t)(
                    functools.partial(start_in, jn, (b - 1) % depth))
                wait_in(b)
                pl.when(t > 0)(functools.partial(wait_out, b))
                compute(j, b)
            return carry

        lax.fori_loop(0, n_rounds, round_body, 0)
        for b in range(depth):
            wait_out(b)

    return sc_kernel


def kernel(h, edge_index, W, b):
    n = h.shape[0]
    e_total = edge_index.shape[1]
    npad = ((n + 127) // 128) * 128
    tbl = _build_table(h, W, b, npad)
    ei = edge_index.astype(jnp.int32)
    chunk = 3200
    score = _make_sc_gather(e_total, npad, chunk)(ei, tbl)
    return score.reshape(e_total, 1)


# speedup vs baseline: 1.0564x; 1.0564x over previous
import functools

import jax
import jax.numpy as jnp
from jax import lax
from jax.experimental import pallas as pl
from jax.experimental.pallas import tpu as pltpu
from jax.experimental.pallas import tpu_sc as plsc

_NC = 2
_NS = 16
_NW = _NC * _NS
_LANES = 16


def _rne_bf16_bits(x):
    u = lax.bitcast_convert_type(x, jnp.uint32)
    return u + jnp.uint32(0x7FFF) + ((u >> jnp.uint32(16)) & jnp.uint32(1))


def _proj_pack_body(w_ref, ht_ref, o_ref):
    ps = w_ref[10] * jnp.ones_like(ht_ref[0:1, :])
    pd = jnp.zeros_like(ps)
    for d in range(5):
        row = ht_ref[d : d + 1, :]
        ps = ps + w_ref[d] * row
        pd = pd + w_ref[5 + d] * row
    lo = _rne_bf16_bits(ps) >> jnp.uint32(16)
    hi = _rne_bf16_bits(pd) & jnp.uint32(0xFFFF0000)
    o_ref[...] = lax.bitcast_convert_type(hi | lo, jnp.int32).reshape(-1)


def _build_table(h, W, b, npad):
    n = h.shape[0]
    ht = jnp.zeros((8, npad), jnp.float32).at[:5, :n].set(h.T)
    wsm = jnp.concatenate(
        [W.reshape(-1), b.reshape(-1), jnp.zeros((5,), jnp.float32)]
    )
    return pl.pallas_call(
        _proj_pack_body,
        out_shape=jax.ShapeDtypeStruct((npad,), jnp.int32),
        in_specs=[
            pl.BlockSpec(memory_space=pltpu.SMEM),
            pl.BlockSpec(memory_space=pltpu.VMEM),
        ],
        out_specs=pl.BlockSpec(memory_space=pltpu.VMEM),
    )(wsm, ht)


def _make_sc_gather(e_total, npad, chunk):
    tot_chunks = e_total // chunk
    depth = 4
    n_t = -(-tot_chunks // _NW)
    n_t = -(-n_t // depth) * depth
    n_rounds = n_t // depth
    n_vec = chunk // _LANES
    mesh = plsc.VectorSubcoreMesh(core_axis_name="c", subcore_axis_name="s")

    @functools.partial(
        pl.kernel,
        mesh=mesh,
        compiler_params=pltpu.CompilerParams(needs_layout_passes=False),
        out_type=jax.ShapeDtypeStruct((e_total,), jnp.float32),
        scratch_types=(
            [pltpu.VMEM((npad,), jnp.int32)]
            + [pltpu.VMEM((2, chunk), jnp.int32) for _ in range(depth)]
            + [pltpu.VMEM((chunk,), jnp.float32) for _ in range(depth)]
            + [pltpu.SemaphoreType.DMA for _ in range(2 * depth + 1)]
        ),
    )
    def sc_kernel(ei_hbm, tbl_hbm, out_hbm, tbl_v, *bufs_flat):
        idx_bufs = bufs_flat[:depth]
        out_bufs = bufs_flat[depth:2 * depth]
        in_sems = bufs_flat[2 * depth:3 * depth]
        out_sems = bufs_flat[3 * depth:4 * depth]
        tbl_sem = bufs_flat[4 * depth]
        wid = lax.axis_index("s") * _NC + lax.axis_index("c")

        def chunk_off(j):
            return lax.rem(wid + _NW * j, tot_chunks) * chunk

        def start_in(j, b):
            pltpu.async_copy(ei_hbm.at[:, pl.ds(chunk_off(j), chunk)],
                             idx_bufs[b], in_sems[b])

        def wait_in(b):
            pltpu.make_async_copy(ei_hbm.at[:, pl.ds(0, chunk)],
                                  idx_bufs[b], in_sems[b]).wait()

        def wait_out(b):
            pltpu.make_async_copy(out_bufs[b], out_hbm.at[pl.ds(0, chunk)],
                                  out_sems[b]).wait()

        def compute(j, b):
            idx_v, out_v = idx_bufs[b], out_bufs[b]

            def vec_body(i):
                s = idx_v[0, pl.ds(i * _LANES, _LANES)]
                d = idx_v[1, pl.ds(i * _LANES, _LANES)]
                g_s = plsc.load_gather(tbl_v, [s])
                g_d = plsc.load_gather(tbl_v, [d])
                ps = plsc.bitcast(g_s << jnp.int32(16), jnp.float32)
                pd = plsc.bitcast(g_d & jnp.int32(-65536), jnp.float32)
                out_v[pl.ds(i * _LANES, _LANES)] = ps + pd

            plsc.parallel_loop(0, n_vec, 1, unroll=16)(vec_body)
            pltpu.async_copy(out_v, out_hbm.at[pl.ds(chunk_off(j), chunk)],
                             out_sems[b])

        pltpu.async_copy(tbl_hbm, tbl_v, tbl_sem)
        for b in range(depth - 1):
            start_in(b, b)
        pltpu.make_async_copy(tbl_hbm, tbl_v, tbl_sem).wait()

        def round_body(t, carry):
            for b in range(depth):
                j = depth * t + b
                jn = j + depth - 1
                pl.when(jn < n_t)(
                    functools.partial(start_in, jn, (b - 1) % depth))
                wait_in(b)
                pl.when(t > 0)(functools.partial(wait_out, b))
                compute(j, b)
            return carry

        lax.fori_loop(0, n_rounds, round_body, 0)
        for b in range(depth):
            wait_out(b)

    return sc_kernel


def kernel(h, edge_index, W, b):
    n = h.shape[0]
    e_total = edge_index.shape[1]
    npad = ((n + 127) // 128) * 128
    tbl = _build_table(h, W, b, npad)
    ei = edge_index.astype(jnp.int32)
    chunk = 2560
    score = _make_sc_gather(e_total, npad, chunk)(ei, tbl)
    return score.reshape(e_total, 1)
